# Initial kernel scaffold; baseline (speedup 1.0000x reference)
import jax
import jax.numpy as jnp
from jax.experimental import pallas as pl


def kernel(x, edge_index, W1, b1, W2, b2):
    raise NotImplementedError("write your pallas kernel here")


# speedup vs baseline: 16.6484x; 16.6484x over previous
import functools

import jax
import jax.numpy as jnp
import numpy as np
from jax import lax
from jax.experimental import pallas as pl
from jax.experimental.pallas import tpu as pltpu
from jax.experimental.pallas import tpu_sc as plsc

N = 10000
NPAD = 10240
DIN = 128
DH = 128
DOUT = 64
E = 320000
NW = 32
CHUNK = 64
NCHUNK = 160
EPT = NCHUNK * CHUNK
EPAD = EPT * NW
RPS = NPAD // 16
BR = 256
GRID = NPAD // BR

_mesh = plsc.VectorSubcoreMesh(core_axis_name="c", subcore_axis_name="s")


def _mk_perm(D):
    p = np.empty((D,), np.int32)
    for k in range(D // 32):
        for i in range(16):
            p[32 * k + 2 * i] = 32 * k + i
            p[32 * k + 2 * i + 1] = 32 * k + 16 + i
    return p


_PERM128 = _mk_perm(DH)
_PERM64 = _mk_perm(DOUT)


@functools.partial(
    pl.kernel,
    out_type=jax.ShapeDtypeStruct((NW * NPAD,), jnp.float32),
    mesh=_mesh,
    compiler_params=pltpu.CompilerParams(needs_layout_passes=False),
    scratch_types=[
        pltpu.VMEM((EPT,), jnp.int32),
        pltpu.VMEM((NPAD,), jnp.float32),
    ],
)
def _deg_hist(dst_hbm, out_hbm, didx_v, hist_v):
    c = lax.axis_index("c")
    s = lax.axis_index("s")
    wid = s * 2 + c

    zeros16 = jnp.zeros((16,), jnp.float32)

    def zbody(i, carry):
        hist_v[pl.ds(i * 16, 16)] = zeros16
        return carry

    lax.fori_loop(0, NPAD // 16, zbody, 0)

    pltpu.sync_copy(dst_hbm.at[pl.ds(wid * EPT, EPT)], didx_v)

    ones16 = jnp.ones((16,), jnp.float32)

    def body(i, carry):
        idx = didx_v[pl.ds(i * 16, 16)]
        plsc.addupdate_scatter(hist_v, [idx], ones16)
        return carry

    lax.fori_loop(0, EPT // 16, body, 0)

    pltpu.sync_copy(hist_v, out_hbm.at[pl.ds(wid * NPAD, NPAD)])


NB = 32
NBATCH = NCHUNK // NB


def _make_agg(D):
    NBUF = 4 if D >= 128 else 8
    UNR = 2 if D >= 128 else 4

    @functools.partial(
        pl.kernel,
        out_type=jax.ShapeDtypeStruct((2 * NPAD, D), jnp.float32),
        mesh=_mesh,
        compiler_params=pltpu.CompilerParams(
            needs_layout_passes=False, use_tc_tiling_on_sc=False),
        scratch_types=(
            [pltpu.VMEM((2 * NB, CHUNK), jnp.int32) for _ in range(2)]
            + [pltpu.VMEM((CHUNK, D), jnp.bfloat16) for _ in range(NBUF)]
            + [pltpu.VMEM((CHUNK, D), jnp.float32) for _ in range(2)]
            + [pltpu.VMEM_SHARED((NPAD, D), jnp.float32)]
            + [pltpu.SemaphoreType.DMA for _ in range(4 + NBUF)]
        ),
    )
    def agg(y_hbm, eidx_hbm, out_hbm, *refs):
        ib = refs[0:2]
        rows = refs[2:2 + NBUF]
        stage = refs[2 + NBUF:4 + NBUF]
        acc = refs[4 + NBUF]
        sem_ib = refs[5 + NBUF:7 + NBUF]
        sem_s = refs[7 + NBUF:9 + NBUF]
        sem_g = refs[9 + NBUF:9 + 2 * NBUF]

        c = lax.axis_index("c")
        s = lax.axis_index("s")
        wid = s * 2 + c

        z16 = jnp.zeros((16,), jnp.float32)
        zi16 = jnp.zeros((16,), jnp.int32)

        def zrow(row, carry):
            for q in range(2):
                for k in range(D // 16):
                    stage[q][row, pl.ds(16 * k, 16)] = z16
            return carry

        lax.fori_loop(0, CHUNK, zrow, 0)
        for k in range(CHUNK // 16):
            ib[1][0, pl.ds(16 * k, 16)] = zi16
        for q in range(RPS // CHUNK):
            pltpu.sync_copy(stage[0],
                            acc.at[pl.ds(s * RPS + q * CHUNK, CHUNK)])
        plsc.subcore_barrier()

        pltpu.async_copy(eidx_hbm.at[wid * NBATCH], ib[0], sem_ib[0])
        for q in range(2):
            pltpu.async_copy(stage[q], acc.at[ib[1].at[0]], sem_s[q],
                             add=True)

        def batch(b, carry):
            for p in range(2):
                @pl.when(lax.rem(b, 2) == p)
                def _(p=p):
                    cur = ib[p]
                    pltpu.make_async_copy(eidx_hbm.at[wid * NBATCH + b],
                                          cur, sem_ib[p]).wait()

                    @pl.when(b < NBATCH - 1)
                    def _():
                        pltpu.async_copy(eidx_hbm.at[wid * NBATCH + b + 1],
                                         ib[1 - p], sem_ib[1 - p])

                    for u in range(NBUF):
                        pltpu.async_copy(y_hbm.at[cur.at[u]],
                                         rows[u], sem_g[u])
                    for u in range(NB):
                        r = u % NBUF
                        q = u % 2
                        pltpu.make_async_copy(y_hbm.at[cur.at[u]],
                                              rows[r], sem_g[r]).wait()
                        pltpu.make_async_copy(stage[q], acc.at[ib[1].at[0]],
                                              sem_s[q]).wait()
                        rb = rows[r]
                        st = stage[q]

                        def conv(h, carry2, rb=rb, st=st):
                            for dr in range(UNR):
                                row = h * UNR + dr
                                for k in range(D // 32):
                                    v = rb[row, pl.ds(32 * k, 32)]
                                    a, b_ = plsc.unpack(
                                        v,
                                        format=plsc.PackFormat.INTERLEAVED)
                                    st[row, pl.ds(32 * k, 16)] = a
                                    st[row, pl.ds(32 * k + 16, 16)] = b_
                            return carry2

                        lax.fori_loop(0, CHUNK // UNR, conv, 0)
                        pltpu.async_copy(st, acc.at[cur.at[NB + u]],
                                         sem_s[q], add=True)
                        if u + NBUF < NB:
                            pltpu.async_copy(y_hbm.at[cur.at[u + NBUF]],
                                             rows[r], sem_g[r])
            return carry

        lax.fori_loop(0, NBATCH, batch, 0)
        for q in range(2):
            pltpu.make_async_copy(stage[q], acc.at[ib[1].at[0]],
                                  sem_s[q]).wait()

        plsc.subcore_barrier()
        pltpu.sync_copy(acc.at[pl.ds(s * RPS, RPS)],
                        out_hbm.at[pl.ds(c * NPAD + s * RPS, RPS)])

    return agg


_agg128 = _make_agg(DH)
_agg64 = _make_agg(DOUT)


def _dis(hist2d):
    def body(h_ref, o_ref):
        deg = jnp.sum(h_ref[...], axis=0, keepdims=True) + 1.0
        r = lax.rsqrt(deg)
        col = lax.broadcasted_iota(jnp.int32, (1, NPAD), 1)
        o_ref[...] = jnp.where(col < N, r, 0.0)

    return pl.pallas_call(
        body,
        out_shape=jax.ShapeDtypeStruct((1, NPAD), jnp.float32),
    )(hist2d)


def _y1(x_pad, W1, W1p, dis_col):
    def body(x_ref, w_ref, wp_ref, d_ref, o_ref, ob_ref):
        x = x_ref[...]
        d = d_ref[...]
        o_ref[...] = jnp.dot(x, w_ref[...],
                             preferred_element_type=jnp.float32) * d
        ob_ref[...] = (jnp.dot(x, wp_ref[...],
                               preferred_element_type=jnp.float32)
                       * d).astype(jnp.bfloat16)

    return pl.pallas_call(
        body,
        grid=(GRID,),
        in_specs=[
            pl.BlockSpec((BR, DIN), lambda i: (i, 0)),
            pl.BlockSpec((DIN, DH), lambda i: (0, 0)),
            pl.BlockSpec((DIN, DH), lambda i: (0, 0)),
            pl.BlockSpec((BR, 1), lambda i: (i, 0)),
        ],
        out_specs=[
            pl.BlockSpec((BR, DH), lambda i: (i, 0)),
            pl.BlockSpec((BR, DH), lambda i: (i, 0)),
        ],
        out_shape=[
            jax.ShapeDtypeStruct((NPAD, DH), jnp.float32),
            jax.ShapeDtypeStruct((NPAD, DH), jnp.bfloat16),
        ],
    )(x_pad, W1, W1p, dis_col)


def _y2(agg0, agg1, y1, dis_col, b1, W2, W2p):
    def body(a0_ref, a1_ref, y_ref, d_ref, b_ref, w_ref, wp_ref,
             o_ref, ob_ref):
        d = d_ref[...]
        h = (a0_ref[...] + a1_ref[...] + y_ref[...]) * d + b_ref[...]
        h = jnp.maximum(h, 0.0)
        o_ref[...] = jnp.dot(h, w_ref[...],
                             preferred_element_type=jnp.float32) * d
        ob_ref[...] = (jnp.dot(h, wp_ref[...],
                               preferred_element_type=jnp.float32)
                       * d).astype(jnp.bfloat16)

    return pl.pallas_call(
        body,
        grid=(GRID,),
        in_specs=[
            pl.BlockSpec((BR, DH), lambda i: (i, 0)),
            pl.BlockSpec((BR, DH), lambda i: (i, 0)),
            pl.BlockSpec((BR, DH), lambda i: (i, 0)),
            pl.BlockSpec((BR, 1), lambda i: (i, 0)),
            pl.BlockSpec((1, DH), lambda i: (0, 0)),
            pl.BlockSpec((DH, DOUT), lambda i: (0, 0)),
            pl.BlockSpec((DH, DOUT), lambda i: (0, 0)),
        ],
        out_specs=[
            pl.BlockSpec((BR, DOUT), lambda i: (i, 0)),
            pl.BlockSpec((BR, DOUT), lambda i: (i, 0)),
        ],
        out_shape=[
            jax.ShapeDtypeStruct((NPAD, DOUT), jnp.float32),
            jax.ShapeDtypeStruct((NPAD, DOUT), jnp.bfloat16),
        ],
    )(agg0, agg1, y1, dis_col, b1, W2, W2p)


def _final(agg0, agg1, y2, dis_col, b2):
    def body(a0_ref, a1_ref, y_ref, d_ref, b_ref, o_ref):
        v = (a0_ref[...] + a1_ref[...] + y_ref[...]) * d_ref[...] + b_ref[...]
        m = jnp.max(v, axis=1, keepdims=True)
        lse = jnp.log(jnp.sum(jnp.exp(v - m), axis=1, keepdims=True)) + m
        o_ref[...] = v - lse

    return pl.pallas_call(
        body,
        grid=(GRID,),
        in_specs=[
            pl.BlockSpec((BR, DOUT), lambda i: (i, 0)),
            pl.BlockSpec((BR, DOUT), lambda i: (i, 0)),
            pl.BlockSpec((BR, DOUT), lambda i: (i, 0)),
            pl.BlockSpec((BR, 1), lambda i: (i, 0)),
            pl.BlockSpec((1, DOUT), lambda i: (0, 0)),
        ],
        out_specs=pl.BlockSpec((BR, DOUT), lambda i: (i, 0)),
        out_shape=jax.ShapeDtypeStruct((NPAD, DOUT), jnp.float32),
    )(agg0, agg1, y2, dis_col, b2)


def kernel(x, edge_index, W1, b1, W2, b2):
    src = edge_index[0].astype(jnp.int32)
    dst = edge_index[1].astype(jnp.int32)
    src_pad = jnp.concatenate([src, jnp.zeros((EPAD - E,), jnp.int32)])
    dst_pad = jnp.concatenate([dst, jnp.full((EPAD - E,), NPAD - 1, jnp.int32)])
    x_pad = jnp.concatenate([x, jnp.zeros((NPAD - N, DIN), x.dtype)])

    src_r = src_pad.reshape(NW, NBATCH, NB, CHUNK)
    dst_r = dst_pad.reshape(NW, NBATCH, NB, CHUNK)
    eidx = jnp.concatenate([src_r, dst_r], axis=2).reshape(
        NW * NBATCH, 2 * NB, CHUNK)

    hist = _deg_hist(dst_pad).reshape(NW, NPAD)
    dis_col = _dis(hist).reshape(NPAD, 1)

    W1p = W1[:, _PERM128]
    W2p = W2[:, _PERM64]
    y1, y1b = _y1(x_pad, W1, W1p, dis_col)
    agg1 = _agg128(y1b, eidx).reshape(2, NPAD, DH)

    y2, y2b = _y2(agg1[0], agg1[1], y1, dis_col, b1.reshape(1, DH), W2, W2p)
    agg2 = _agg64(y2b, eidx).reshape(2, NPAD, DOUT)

    out = _final(agg2[0], agg2[1], y2, dis_col, b2.reshape(1, DOUT))
    return out[:N]
